# baseline (speedup 1.0000x reference)
import functools

import jax
import jax.numpy as jnp
from jax import lax
from jax.experimental import pallas as pl
from jax.experimental.pallas import tpu as pltpu
from jax.experimental.pallas import tpu_sc as plsc

B = 16384
D = 64
NC = 2
NS = 16
L = 16
NW = NC * NS
BPW = B // NW
CH = 128
CWID = 384
SLOTS = 544
IDPAD = B + (SLOTS - BPW) + L
TRASH = B

_mesh = plsc.VectorSubcoreMesh(core_axis_name="c", subcore_axis_name="s")
_params = pltpu.CompilerParams(use_tc_tiling_on_sc=True, needs_layout_passes=False)


def _make_extract(cmaxstart, cwid, nbuf):
    @functools.partial(
        pl.kernel,
        mesh=_mesh,
        out_type=jax.ShapeDtypeStruct((B + 8, 128), jnp.float32),
        scratch_types=[
            pltpu.VMEM((nbuf * 64, cwid), jnp.float32),
            pltpu.VMEM((SLOTS + L,), jnp.int32),
            pltpu.VMEM((4, CH), jnp.int32),
            pltpu.VMEM((SLOTS, 128), jnp.float32),
            pltpu.SemaphoreType.DMA,
            pltpu.SemaphoreType.DMA,
            pltpu.SemaphoreType.DMA,
            pltpu.SemaphoreType.DMA,
        ],
        compiler_params=_params,
    )
    def _extract(ids_hbm, pos_hbm, tab_hbm, emb_hbm,
                 chunk, idv, psc, embv, sem, csem0, csem1, csem2):
        wid = lax.axis_index("s") * NC + lax.axis_index("c")
        base = wid * BPW
        pltpu.sync_copy(ids_hbm.at[pl.ds(base, SLOTS + L)], idv)
        for j in range(4):
            pltpu.sync_copy(pos_hbm.at[pl.ds(base + j * CH, CH)], psc.at[j])

        d16 = lax.broadcasted_iota(jnp.int32, (L,), 0)

        def chunk_start(r):
            u0 = idv[pl.ds(r, L)][0]
            return jnp.minimum((u0 // cwid) * cwid, jnp.int32(cmaxstart))

        csems = (csem0, csem1, csem2)

        def fire(q, cstart):
            pltpu.async_copy(
                tab_hbm.at[:, pl.ds(pl.multiple_of(cstart, 128), cwid)],
                chunk.at[pl.ds(q * 64, 64)], csems[q])

        def drain(q):
            pltpu.make_async_copy(
                tab_hbm.at[:, pl.ds(0, cwid)],
                chunk.at[pl.ds(q * 64, 64)], csems[q]).wait()

        def extract(rowbase, r, uv0, cstart):
            for j in range(L):
                uj = uv0[j]
                cc = jnp.minimum(jnp.maximum(uj - cstart, 0), cwid - 1)
                ccv = jnp.zeros((L,), jnp.int32) + cc
                for k in range(4):
                    g = plsc.load_gather(chunk, [rowbase + d16 + k * L, ccv])
                    embv[r + j, pl.ds(k * L, L)] = g

        if nbuf == 3:
            def nxt(c):
                return jnp.minimum(c + cwid, jnp.int32(cmaxstart))

            c0 = chunk_start(jnp.int32(0))
            fire(0, c0)
            fire(1, nxt(c0))
            fire(2, nxt(nxt(c0)))

            def body3(carry):
                r, q, cstart, fresh = carry
                s1 = jnp.where(q == 2, 0, q + 1)
                s2 = jnp.where(s1 == 2, 0, s1 + 1)
                uv0 = idv[pl.ds(r, L)]
                m = (uv0 < cstart + cwid).astype(jnp.int32)
                n = jnp.sum(m)

                for k in range(3):
                    @pl.when((fresh == 1) & (q == k))
                    def _(k=k):
                        drain(k)

                extract(q * 64, r, uv0, cstart)

                csn = chunk_start(r + n)
                moved = csn != cstart
                jump = moved & (csn != nxt(cstart))

                for k in range(3):
                    @pl.when(jump & (s1 == k))
                    def _(k=k):
                        drain(k)
                        fire(k, csn)

                for k in range(3):
                    @pl.when(jump & (s2 == k))
                    def _(k=k):
                        drain(k)
                        fire(k, nxt(csn))

                for k in range(3):
                    @pl.when(moved & (q == k))
                    def _(k=k):
                        fire(k, nxt(nxt(csn)))

                qn = jnp.where(moved, s1, q)
                cn = jnp.where(moved, csn, cstart)
                return r + n, qn, cn, moved.astype(jnp.int32)

            rf, qf, cf, ff = lax.while_loop(
                lambda c: c[0] < BPW, body3,
                (jnp.int32(0), jnp.int32(0), c0, jnp.int32(1)))

            t1 = jnp.where(qf == 2, 0, qf + 1)
            t2 = jnp.where(t1 == 2, 0, t1 + 1)
            for k in range(3):
                @pl.when((ff == 1) & (qf == k))
                def _(k=k):
                    drain(k)
            for k in range(3):
                @pl.when(t1 == k)
                def _(k=k):
                    drain(k)
            for k in range(3):
                @pl.when(t2 == k)
                def _(k=k):
                    drain(k)
        elif nbuf == 1:
            cs0 = chunk_start(jnp.int32(0))
            fire(0, cs0)
            drain(0)

            def body1(carry):
                r, cstart = carry
                uv0 = idv[pl.ds(r, L)]
                m = (uv0 < cstart + cwid).astype(jnp.int32)
                n = jnp.sum(m)
                csn = chunk_start(r + n)
                extract(0, r, uv0, cstart)

                @pl.when(csn != cstart)
                def _():
                    fire(0, csn)
                    drain(0)

                return r + n, csn

            lax.while_loop(lambda c: c[0] < BPW, body1, (jnp.int32(0), cs0))
        else:
            cs0 = chunk_start(jnp.int32(0))
            fire(0, cs0)

            def body2(carry):
                r, p, cstart, fresh = carry
                uv0 = idv[pl.ds(r, L)]
                m = (uv0 < cstart + cwid).astype(jnp.int32)
                n = jnp.sum(m)
                csn = chunk_start(r + n)
                moved = csn != cstart

                @pl.when(moved & (p == 0))
                def _():
                    fire(1, csn)

                @pl.when(moved & (p == 1))
                def _():
                    fire(0, csn)

                @pl.when((fresh == 1) & (p == 0))
                def _():
                    drain(0)

                @pl.when((fresh == 1) & (p == 1))
                def _():
                    drain(1)

                extract(p * 64, r, uv0, cstart)
                pn = jnp.where(moved, 1 - p, p)
                return r + n, pn, csn, moved.astype(jnp.int32)

            rf, pf, _, ff = lax.while_loop(
                lambda c: c[0] < BPW, body2,
                (jnp.int32(0), jnp.int32(0), cs0, jnp.int32(1)))

            @pl.when((ff == 1) & (pf == 0))
            def _():
                drain(0)

            @pl.when((ff == 1) & (pf == 1))
            def _():
                drain(1)

        for j in range(4):
            pltpu.async_copy(embv.at[pl.ds(j * CH, CH)],
                             emb_hbm.at[psc.at[j]], sem)
        for j in range(4):
            pltpu.make_async_copy(emb_hbm.at[pl.ds(0, CH)],
                                  embv.at[pl.ds(0, CH)], sem).wait()

    return _extract


_extract_user = _make_extract(1000064 - 256, 256, 3)
_extract_book = _make_extract(100096 - 768, 768, 1)


@functools.partial(
    pl.kernel,
    mesh=_mesh,
    out_type=jax.ShapeDtypeStruct((B,), jnp.float32),
    scratch_types=[
        pltpu.VMEM((BPW // 2, 128), jnp.float32),
        pltpu.VMEM((BPW // 2, 128), jnp.float32),
        pltpu.VMEM((BPW,), jnp.float32),
    ],
    compiler_params=_params,
)
def _dot(uemb_hbm, bemb_hbm, out_hbm, urows, brows, outv):
    wid = lax.axis_index("s") * NC + lax.axis_index("c")
    lane = lax.broadcasted_iota(jnp.int32, (L,), 0)
    HB = BPW // 2

    for h in range(2):
        hb = h * HB
        pltpu.sync_copy(uemb_hbm.at[pl.ds(wid * BPW + hb, HB), :], urows)
        pltpu.sync_copy(bemb_hbm.at[pl.ds(wid * BPW + hb, HB), :], brows)

        def group(g, carry):
            r0 = g * L
            acc = jnp.zeros((L,), jnp.float32)
            for j in range(L):
                r = r0 + j
                s = jnp.zeros((L,), jnp.float32)
                for k in range(D // L):
                    s = s + (urows[r, pl.ds(k * L, L)]
                             * brows[r, pl.ds(k * L, L)])
                acc = jnp.where(lane == j, jnp.sum(s), acc)
            outv[pl.ds(hb + r0, L)] = acc
            return carry

        lax.fori_loop(0, HB // L, group, 0, unroll=False)

    pltpu.sync_copy(outv, out_hbm.at[pl.ds(wid * BPW, BPW)])


def _sorted_ids(ids):
    iota = lax.broadcasted_iota(jnp.int32, (B,), 0)
    ids_sorted, perm = lax.sort((ids, iota), num_keys=1)
    npad = IDPAD - B
    ids_pad = jnp.concatenate(
        [ids_sorted, jnp.full((npad,), jnp.int32(0x7FFFFFFF))])
    pos_pad = jnp.concatenate(
        [perm, jnp.full((npad,), jnp.int32(TRASH))])
    return ids_pad, pos_pad


@jax.jit
def kernel(user_ids, book_ids, user_table, book_table):
    uid = user_ids.reshape(B)
    bid = book_ids.reshape(B)
    up, upos = _sorted_ids(uid)
    bp, bpos = _sorted_ids(bid)
    uemb = _extract_user(up, upos, user_table.T)
    bemb = _extract_book(bp, bpos, book_table.T)
    out = _dot(uemb, bemb)
    return out.reshape(B, 1)
